# baseline (speedup 1.0000x reference)
import functools

import jax
import jax.numpy as jnp
from jax import lax
from jax.experimental import pallas as pl
from jax.experimental.pallas import tpu as pltpu
from jax.experimental.pallas import tpu_sc as plsc

N = 10000
NH = 256
E = 160000

NC = 2
NS = 16
NW = NC * NS

CHUNK = 3200
SUBB = 128
GRP_PER_SUBB = SUBB // 16
ROWS_BASE = 312
BIG_WORKERS = 2
ROWS_BIG = 320
ACC_ROWS = ROWS_BIG


def _sc_spmm_body(dst_hbm, src_hbm, val_hbm, table_hbm, out_hbm,
                  dbuf, sbuf, vbuf, rows, acc, sem):
    wid = lax.axis_index("c") * NS + lax.axis_index("s")
    lo = wid * ROWS_BASE + 8 * jnp.minimum(wid, BIG_WORKERS)
    hi = lo + jnp.where(wid < BIG_WORKERS, ROWS_BIG, ROWS_BASE)

    zf = jnp.zeros((16,), jnp.float32)
    zi = jnp.zeros((16,), jnp.int32)

    def zero_body(r, _):
        for k in range(NH // 16):
            acc[r, pl.ds(k * 16, 16)] = zf
        return 0
    lax.fori_loop(0, ACC_ROWS, zero_body, 0)

    def chunk_body(ch, _):
        off = ch * CHUNK
        c1 = pltpu.async_copy(dst_hbm.at[pl.ds(off, CHUNK)],
                              dbuf.at[pl.ds(0, CHUNK)], sem)
        c2 = pltpu.async_copy(src_hbm.at[pl.ds(off, CHUNK)],
                              sbuf.at[pl.ds(0, CHUNK)], sem)
        c3 = pltpu.async_copy(val_hbm.at[pl.ds(off, CHUNK)],
                              vbuf.at[pl.ds(0, CHUNK)], sem)
        c1.wait()
        c2.wait()
        c3.wait()

        def comp_body(j, cnt):
            d = dbuf[pl.ds(j * 16, 16)]
            s = sbuf[pl.ds(j * 16, 16)]
            v = vbuf[pl.ds(j * 16, 16)]
            m = (d >= lo) & (d < hi)
            mi = m.astype(jnp.int32)
            pos = cnt + plsc.cumsum(mi) - 1
            plsc.store_scatter(dbuf, [pos], d - lo, mask=m)
            plsc.store_scatter(sbuf, [pos], s, mask=m)
            plsc.store_scatter(vbuf, [pos], v, mask=m)
            return cnt + jnp.sum(mi, axis=0)
        cnt = lax.fori_loop(0, CHUNK // 16, comp_body, 0)

        dbuf[pl.ds(cnt, 16)] = zi
        sbuf[pl.ds(cnt, 16)] = zi
        vbuf[pl.ds(cnt, 16)] = zf

        ngroups = (cnt + 15) // 16
        nbatch = (cnt + SUBB - 1) // SUBB

        def sb_body(b, _):
            pltpu.async_copy(
                table_hbm.at[sbuf.at[pl.ds(b * SUBB, SUBB)]], rows, sem
            ).wait()

            def grp_body(g, _):
                base = g * 16
                dl = dbuf[pl.ds(base, 16)]
                v = vbuf[pl.ds(base, 16)]
                e = lax.broadcasted_iota(jnp.int32, (16,), 0) + (base - b * SUBB)
                for c in range(NH):
                    cv = jnp.full((16,), c, jnp.int32)
                    col = plsc.load_gather(rows, [e, cv])
                    plsc.addupdate_scatter(acc, [dl, cv], col * v)
                return 0
            lax.fori_loop(b * GRP_PER_SUBB,
                          jnp.minimum((b + 1) * GRP_PER_SUBB, ngroups),
                          grp_body, 0)
            return 0
        lax.fori_loop(0, nbatch, sb_body, 0)
        return 0

    lax.fori_loop(0, E // CHUNK, chunk_body, 0)

    @pl.when(wid < BIG_WORKERS)
    def _():
        pltpu.sync_copy(acc.at[pl.ds(0, ROWS_BIG), :],
                        out_hbm.at[pl.ds(lo, ROWS_BIG), :])

    @pl.when(wid >= BIG_WORKERS)
    def _():
        pltpu.sync_copy(acc.at[pl.ds(0, ROWS_BASE), :],
                        out_hbm.at[pl.ds(lo, ROWS_BASE), :])


@functools.partial(jax.jit, static_argnums=())
def _spmm_sc(dst, src, val, table):
    mesh = plsc.VectorSubcoreMesh(core_axis_name="c", subcore_axis_name="s")
    f = pl.kernel(
        _sc_spmm_body,
        out_type=jax.ShapeDtypeStruct((N, NH), jnp.float32),
        mesh=mesh,
        compiler_params=pltpu.CompilerParams(needs_layout_passes=False),
        scratch_types=[
            pltpu.VMEM((CHUNK + 16,), jnp.int32),
            pltpu.VMEM((CHUNK + 16,), jnp.int32),
            pltpu.VMEM((CHUNK + 16,), jnp.float32),
            pltpu.VMEM((SUBB, NH), jnp.float32),
            pltpu.VMEM((ACC_ROWS, NH), jnp.float32),
            pltpu.SemaphoreType.DMA,
        ],
    )
    return f(dst, src, val, table)


def _stage_b_body(r1_ref, r2_ref, w2_ref, x1_ref, x2_ref, h1_ref, h2_ref):
    w2 = w2_ref[...]
    for r_ref, x_ref, h_ref in ((r1_ref, x1_ref, h1_ref),
                                (r2_ref, x2_ref, h2_ref)):
        x = jnp.maximum(r_ref[...], 0.0)
        nrm = jnp.sqrt(jnp.sum(x * x, axis=1, keepdims=True))
        xn = x / jnp.maximum(nrm, 1e-12)
        x_ref[...] = xn
        h_ref[...] = jnp.dot(xn, w2, preferred_element_type=jnp.float32)


def _stage_b(r1, r2, W2):
    blk = 1000
    grid = (N // blk,)
    row_spec = pl.BlockSpec((blk, NH), lambda i: (i, 0))
    full_spec = pl.BlockSpec((NH, NH), lambda i: (0, 0))
    return pl.pallas_call(
        _stage_b_body,
        grid=grid,
        in_specs=[row_spec, row_spec, full_spec],
        out_specs=[row_spec] * 4,
        out_shape=[jax.ShapeDtypeStruct((N, NH), jnp.float32)] * 4,
    )(r1, r2, W2)


def _stage_d_body(r3_ref, r4_ref, x1_ref, x2_ref,
                  w1_ref, b1_ref, w2_ref, b2_ref, w3_ref, b3_ref, out_ref):
    w1, b1 = w1_ref[...], b1_ref[...]
    w2, b2 = w2_ref[...], b2_ref[...]
    w3, b3 = w3_ref[...], b3_ref[...]

    def mlp(t):
        h = jnp.maximum(jnp.dot(t, w1, preferred_element_type=jnp.float32) + b1, 0.0)
        h = jnp.maximum(jnp.dot(h, w2, preferred_element_type=jnp.float32) + b2, 0.0)
        return jnp.dot(h, w3, preferred_element_type=jnp.float32) + b3

    y1 = jnp.maximum(r3_ref[...], 0.0)
    y2 = jnp.maximum(r4_ref[...], 0.0)
    s1 = mlp(x1_ref[...]) + mlp(y1)
    s2 = mlp(x2_ref[...]) + mlp(y2)
    out_ref[...] = s1 * s2


def _stage_d(r3, r4, x1, x2, w1, b1, w2, b2, w3, b3):
    blk = 1000
    grid = (N // blk,)
    row_spec = pl.BlockSpec((blk, NH), lambda i: (i, 0))
    return pl.pallas_call(
        _stage_d_body,
        grid=grid,
        in_specs=[
            row_spec, row_spec, row_spec, row_spec,
            pl.BlockSpec((NH, 2 * NH), lambda i: (0, 0)),
            pl.BlockSpec((1, 2 * NH), lambda i: (0, 0)),
            pl.BlockSpec((2 * NH, 2 * NH), lambda i: (0, 0)),
            pl.BlockSpec((1, 2 * NH), lambda i: (0, 0)),
            pl.BlockSpec((2 * NH, 1), lambda i: (0, 0)),
            pl.BlockSpec((1, 1), lambda i: (0, 0)),
        ],
        out_specs=pl.BlockSpec((blk, 1), lambda i: (i, 0)),
        out_shape=jax.ShapeDtypeStruct((N, 1), jnp.float32),
    )(r3, r4, x1, x2, w1, b1, w2, b2, w3, b3)


def kernel(adj1_indices, adj1_values, adj2_indices, adj2_values,
           W1, W2, mlp_w1, mlp_b1, mlp_w2, mlp_b2, mlp_w3, mlp_b3):
    dst1, src1 = adj1_indices[0], adj1_indices[1]
    dst2, src2 = adj2_indices[0], adj2_indices[1]

    r1 = _spmm_sc(dst1, src1, adj1_values, W1)
    r2 = _spmm_sc(dst2, src2, adj2_values, W1)
    x1, x2, h1, h2 = _stage_b(r1, r2, W2)
    r3 = _spmm_sc(dst1, src1, adj1_values, h1)
    r4 = _spmm_sc(dst2, src2, adj2_values, h2)

    b1 = mlp_b1.reshape(1, -1)
    b2 = mlp_b2.reshape(1, -1)
    b3 = mlp_b3.reshape(1, -1)
    return _stage_d(r3, r4, x1, x2, mlp_w1, b1, mlp_w2, b2, mlp_w3, b3)


# speedup vs baseline: 2.0919x; 2.0919x over previous
import functools

import jax
import jax.numpy as jnp
from jax import lax
from jax.experimental import pallas as pl
from jax.experimental.pallas import tpu as pltpu
from jax.experimental.pallas import tpu_sc as plsc

N = 10000
NH = 256
E = 160000

NC = 2
NS = 16
NW = NC * NS

CHUNK = 3200
SUBB = 128
GRP_PER_SUBB = SUBB // 16
ROWS_BASE = 312
BIG_WORKERS = 2
ROWS_BIG = 320
ACC_ROWS = ROWS_BIG


def _sc_spmm_body(dst_hbm, src_hbm, val_hbm, table_hbm, out_hbm,
                  dbuf, sbuf, vbuf, rows, acc, sem):
    wid = lax.axis_index("c") * NS + lax.axis_index("s")
    lo = wid * ROWS_BASE + 8 * jnp.minimum(wid, BIG_WORKERS)
    hi = lo + jnp.where(wid < BIG_WORKERS, ROWS_BIG, ROWS_BASE)

    zf = jnp.zeros((16,), jnp.float32)
    zi = jnp.zeros((16,), jnp.int32)

    def zero_body(r, _):
        for k in range(NH // 16):
            acc[r, pl.ds(k * 16, 16)] = zf
        return 0
    lax.fori_loop(0, ACC_ROWS, zero_body, 0)

    def chunk_body(ch, _):
        off = ch * CHUNK
        c1 = pltpu.async_copy(dst_hbm.at[pl.ds(off, CHUNK)],
                              dbuf.at[pl.ds(0, CHUNK)], sem)
        c2 = pltpu.async_copy(src_hbm.at[pl.ds(off, CHUNK)],
                              sbuf.at[pl.ds(0, CHUNK)], sem)
        c3 = pltpu.async_copy(val_hbm.at[pl.ds(off, CHUNK)],
                              vbuf.at[pl.ds(0, CHUNK)], sem)
        c1.wait()
        c2.wait()
        c3.wait()

        def comp_body(j, cnt):
            d = dbuf[pl.ds(j * 16, 16)]
            s = sbuf[pl.ds(j * 16, 16)]
            v = vbuf[pl.ds(j * 16, 16)]
            m = (d >= lo) & (d < hi)
            mi = m.astype(jnp.int32)
            pos = cnt + plsc.cumsum(mi) - 1
            plsc.store_scatter(dbuf, [pos], d - lo, mask=m)
            plsc.store_scatter(sbuf, [pos], s, mask=m)
            plsc.store_scatter(vbuf, [pos], v, mask=m)
            return cnt + jnp.sum(mi, axis=0)
        cnt = lax.fori_loop(0, CHUNK // 16, comp_body, 0)

        dbuf[pl.ds(cnt, 16)] = zi
        sbuf[pl.ds(cnt, 16)] = zi
        vbuf[pl.ds(cnt, 16)] = zf

        nbatch = (cnt + SUBB - 1) // SUBB

        def sb_body(b, _):
            pltpu.async_copy(
                table_hbm.at[sbuf.at[pl.ds(b * SUBB, SUBB)]], rows, sem
            ).wait()
            nedge = jnp.minimum(cnt - b * SUBB, SUBB)

            def e_body(i, _):
                dle = dbuf[pl.ds(b * SUBB + i, 16)][0]
                vv = jnp.full((16,), vbuf[pl.ds(b * SUBB + i, 16)][0],
                              jnp.float32)
                for k in range(NH // 16):
                    prod = rows[i, pl.ds(k * 16, 16)] * vv
                    plsc.addupdate(acc.at[dle, pl.ds(k * 16, 16)], prod)
                return 0
            lax.fori_loop(0, nedge, e_body, 0)
            return 0
        lax.fori_loop(0, nbatch, sb_body, 0)
        return 0

    lax.fori_loop(0, E // CHUNK, chunk_body, 0)

    @pl.when(wid < BIG_WORKERS)
    def _():
        pltpu.sync_copy(acc.at[pl.ds(0, ROWS_BIG), :],
                        out_hbm.at[pl.ds(lo, ROWS_BIG), :])

    @pl.when(wid >= BIG_WORKERS)
    def _():
        pltpu.sync_copy(acc.at[pl.ds(0, ROWS_BASE), :],
                        out_hbm.at[pl.ds(lo, ROWS_BASE), :])


@functools.partial(jax.jit, static_argnums=())
def _spmm_sc(dst, src, val, table):
    mesh = plsc.VectorSubcoreMesh(core_axis_name="c", subcore_axis_name="s")
    f = pl.kernel(
        _sc_spmm_body,
        out_type=jax.ShapeDtypeStruct((N, NH), jnp.float32),
        mesh=mesh,
        compiler_params=pltpu.CompilerParams(needs_layout_passes=False),
        scratch_types=[
            pltpu.VMEM((CHUNK + 16,), jnp.int32),
            pltpu.VMEM((CHUNK + 16,), jnp.int32),
            pltpu.VMEM((CHUNK + 16,), jnp.float32),
            pltpu.VMEM((SUBB, NH), jnp.float32),
            pltpu.VMEM((ACC_ROWS, NH), jnp.float32),
            pltpu.SemaphoreType.DMA,
        ],
    )
    return f(dst, src, val, table)


def _stage_b_body(r1_ref, r2_ref, w2_ref, x1_ref, x2_ref, h1_ref, h2_ref):
    w2 = w2_ref[...]
    for r_ref, x_ref, h_ref in ((r1_ref, x1_ref, h1_ref),
                                (r2_ref, x2_ref, h2_ref)):
        x = jnp.maximum(r_ref[...], 0.0)
        nrm = jnp.sqrt(jnp.sum(x * x, axis=1, keepdims=True))
        xn = x / jnp.maximum(nrm, 1e-12)
        x_ref[...] = xn
        h_ref[...] = jnp.dot(xn, w2, preferred_element_type=jnp.float32)


def _stage_b(r1, r2, W2):
    blk = 1000
    grid = (N // blk,)
    row_spec = pl.BlockSpec((blk, NH), lambda i: (i, 0))
    full_spec = pl.BlockSpec((NH, NH), lambda i: (0, 0))
    return pl.pallas_call(
        _stage_b_body,
        grid=grid,
        in_specs=[row_spec, row_spec, full_spec],
        out_specs=[row_spec] * 4,
        out_shape=[jax.ShapeDtypeStruct((N, NH), jnp.float32)] * 4,
    )(r1, r2, W2)


def _stage_d_body(r3_ref, r4_ref, x1_ref, x2_ref,
                  w1_ref, b1_ref, w2_ref, b2_ref, w3_ref, b3_ref, out_ref):
    w1, b1 = w1_ref[...], b1_ref[...]
    w2, b2 = w2_ref[...], b2_ref[...]
    w3, b3 = w3_ref[...], b3_ref[...]

    def mlp(t):
        h = jnp.maximum(jnp.dot(t, w1, preferred_element_type=jnp.float32) + b1, 0.0)
        h = jnp.maximum(jnp.dot(h, w2, preferred_element_type=jnp.float32) + b2, 0.0)
        return jnp.dot(h, w3, preferred_element_type=jnp.float32) + b3

    y1 = jnp.maximum(r3_ref[...], 0.0)
    y2 = jnp.maximum(r4_ref[...], 0.0)
    s1 = mlp(x1_ref[...]) + mlp(y1)
    s2 = mlp(x2_ref[...]) + mlp(y2)
    out_ref[...] = s1 * s2


def _stage_d(r3, r4, x1, x2, w1, b1, w2, b2, w3, b3):
    blk = 1000
    grid = (N // blk,)
    row_spec = pl.BlockSpec((blk, NH), lambda i: (i, 0))
    return pl.pallas_call(
        _stage_d_body,
        grid=grid,
        in_specs=[
            row_spec, row_spec, row_spec, row_spec,
            pl.BlockSpec((NH, 2 * NH), lambda i: (0, 0)),
            pl.BlockSpec((1, 2 * NH), lambda i: (0, 0)),
            pl.BlockSpec((2 * NH, 2 * NH), lambda i: (0, 0)),
            pl.BlockSpec((1, 2 * NH), lambda i: (0, 0)),
            pl.BlockSpec((2 * NH, 1), lambda i: (0, 0)),
            pl.BlockSpec((1, 1), lambda i: (0, 0)),
        ],
        out_specs=pl.BlockSpec((blk, 1), lambda i: (i, 0)),
        out_shape=jax.ShapeDtypeStruct((N, 1), jnp.float32),
    )(r3, r4, x1, x2, w1, b1, w2, b2, w3, b3)


def kernel(adj1_indices, adj1_values, adj2_indices, adj2_values,
           W1, W2, mlp_w1, mlp_b1, mlp_w2, mlp_b2, mlp_w3, mlp_b3):
    dst1, src1 = adj1_indices[0], adj1_indices[1]
    dst2, src2 = adj2_indices[0], adj2_indices[1]

    r1 = _spmm_sc(dst1, src1, adj1_values, W1)
    r2 = _spmm_sc(dst2, src2, adj2_values, W1)
    x1, x2, h1, h2 = _stage_b(r1, r2, W2)
    r3 = _spmm_sc(dst1, src1, adj1_values, h1)
    r4 = _spmm_sc(dst2, src2, adj2_values, h2)

    b1 = mlp_b1.reshape(1, -1)
    b2 = mlp_b2.reshape(1, -1)
    b3 = mlp_b3.reshape(1, -1)
    return _stage_d(r3, r4, x1, x2, mlp_w1, b1, mlp_w2, b2, mlp_w3, b3)
